# baseline (speedup 1.0000x reference)
import functools

import jax
import jax.numpy as jnp
from jax import lax
from jax.experimental import pallas as pl
from jax.experimental.pallas import tpu as pltpu
from jax.experimental.pallas import tpu_sc as plsc

N = 10000
E = 320000
D = 128
H = 256
L = 4

NC = 2
NS = 16
NW = NC * NS
C = 128
K = 80
EPW = K * C
EPAD = NW * EPW
NBUF = 2
NP = 10008
RPT = 624
REM = N - NS * RPT
ZREM = NP - NS * RPT

ROWS_BLK = 1000
NBLK = N // ROWS_BLK


def _seg_sum(h, src_r, dst_r, zeros):
  mesh = plsc.VectorSubcoreMesh(core_axis_name="c", subcore_axis_name="s")

  @functools.partial(
      pl.kernel,
      out_type=jax.ShapeDtypeStruct((NC, N, D), jnp.float32),
      mesh=mesh,
      scratch_types=[
          pltpu.VMEM((NBUF, C), jnp.int32),
          pltpu.VMEM((NBUF, C), jnp.int32),
          pltpu.VMEM((NBUF, C, D), jnp.float32),
          pltpu.VMEM_SHARED((NP, D), jnp.float32),
          [pltpu.SemaphoreType.DMA] * NBUF,
          [pltpu.SemaphoreType.DMA] * NBUF,
          [pltpu.SemaphoreType.DMA] * NBUF,
          [pltpu.SemaphoreType.DMA] * NBUF,
      ],
  )
  def k(h_hbm, src_hbm, dst_hbm, z_hbm, out_hbm, sidx, didx, rows_v,
        agg_sh, gsem, ssem, sisem, disem):
    c = lax.axis_index("c")
    s = lax.axis_index("s")
    pltpu.sync_copy(z_hbm.at[pl.ds(s * RPT, RPT)],
                    agg_sh.at[pl.ds(s * RPT, RPT)])

    @pl.when(s == 0)
    def _():
      pltpu.sync_copy(z_hbm.at[pl.ds(NS * RPT, ZREM)],
                      agg_sh.at[pl.ds(NS * RPT, ZREM)])

    plsc.subcore_barrier()

    def wait_idx(ref, sem, b):
      pltpu.make_async_copy(src_hbm.at[c, s, 0], ref.at[b], sem[b]).wait()

    def wait_rows(sem, b):
      pltpu.make_async_copy(h_hbm.at[pl.ds(0, C)], rows_v.at[b],
                            sem[b]).wait()

    for b in range(NBUF):
      pltpu.async_copy(src_hbm.at[c, s, b], sidx.at[b], sisem[b])
      pltpu.async_copy(dst_hbm.at[c, s, b], didx.at[b], disem[b])
    for b in range(NBUF):
      wait_idx(sidx, sisem, b)
      pltpu.async_copy(h_hbm.at[sidx.at[b]], rows_v.at[b], gsem[b])

    def body(jp, carry):
      j0 = jp * NBUF
      for b in range(NBUF):
        wait_rows(gsem, b)

        @pl.when(j0 + b + NBUF < K)
        def _():
          pltpu.async_copy(src_hbm.at[c, s, j0 + b + NBUF], sidx.at[b],
                           sisem[b])

        wait_idx(didx, disem, b)
        pltpu.async_copy(rows_v.at[b], agg_sh.at[didx.at[b]], ssem[b],
                         add=True)
      for b in range(NBUF):
        wait_rows(ssem, b)

        @pl.when(j0 + b + NBUF < K)
        def _():
          pltpu.async_copy(dst_hbm.at[c, s, j0 + b + NBUF], didx.at[b],
                           disem[b])
          wait_idx(sidx, sisem, b)
          pltpu.async_copy(h_hbm.at[sidx.at[b]], rows_v.at[b], gsem[b])

      return carry

    lax.fori_loop(0, K // NBUF, body, 0)
    plsc.subcore_barrier()
    pltpu.sync_copy(agg_sh.at[pl.ds(s * RPT, RPT)],
                    out_hbm.at[c, pl.ds(s * RPT, RPT)])

    @pl.when(s == 0)
    def _():
      pltpu.sync_copy(agg_sh.at[pl.ds(NS * RPT, REM)],
                      out_hbm.at[c, pl.ds(NS * RPT, REM)])

  return k(h, src_r, dst_r, zeros)


def _lin0(x, W, b):
  def body(x_ref, w_ref, b_ref, o_ref):
    o_ref[...] = jnp.dot(x_ref[...], w_ref[...],
                         preferred_element_type=jnp.float32) + b_ref[...]

  return pl.pallas_call(
      body,
      grid=(NBLK,),
      in_specs=[
          pl.BlockSpec((ROWS_BLK, D), lambda i: (i, 0)),
          pl.BlockSpec((D, D), lambda i: (0, 0)),
          pl.BlockSpec((1, D), lambda i: (0, 0)),
      ],
      out_specs=pl.BlockSpec((ROWS_BLK, D), lambda i: (i, 0)),
      out_shape=jax.ShapeDtypeStruct((N, D), jnp.float32),
  )(x, W, b.reshape(1, D))


def _mlp(h, p0, p1, W1, b1, W2, b2):
  def body(h_ref, p0_ref, p1_ref, w1_ref, b1_ref, w2_ref, b2_ref, o_ref):
    hh = h_ref[...]
    z = hh + p0_ref[...] + p1_ref[...]
    a = jnp.dot(z, w1_ref[...], preferred_element_type=jnp.float32)
    a = jnp.maximum(a + b1_ref[...], 0.0)
    zz = jnp.dot(a, w2_ref[...], preferred_element_type=jnp.float32)
    o_ref[...] = hh + jnp.maximum(zz + b2_ref[...], 0.0)

  return pl.pallas_call(
      body,
      grid=(NBLK,),
      in_specs=[
          pl.BlockSpec((ROWS_BLK, D), lambda i: (i, 0)),
          pl.BlockSpec((ROWS_BLK, D), lambda i: (i, 0)),
          pl.BlockSpec((ROWS_BLK, D), lambda i: (i, 0)),
          pl.BlockSpec((D, H), lambda i: (0, 0)),
          pl.BlockSpec((1, H), lambda i: (0, 0)),
          pl.BlockSpec((H, D), lambda i: (0, 0)),
          pl.BlockSpec((1, D), lambda i: (0, 0)),
      ],
      out_specs=pl.BlockSpec((ROWS_BLK, D), lambda i: (i, 0)),
      out_shape=jax.ShapeDtypeStruct((N, D), jnp.float32),
  )(h, p0, p1, W1, b1.reshape(1, H), W2, b2.reshape(1, D))


def _readout(h, W1, b1, W2, b2):
  def body(h_ref, w1_ref, b1_ref, w2_ref, b2_ref, o_ref, acc_ref):
    i = pl.program_id(0)

    @pl.when(i == 0)
    def _():
      acc_ref[...] = jnp.zeros_like(acc_ref)

    acc_ref[...] += jnp.sum(h_ref[...], axis=0, keepdims=True)

    @pl.when(i == NBLK - 1)
    def _():
      s = acc_ref[...]
      a = jnp.dot(s, w1_ref[...], preferred_element_type=jnp.float32)
      a = jnp.maximum(a + b1_ref[...], 0.0)
      o = jnp.dot(a, w2_ref[...], preferred_element_type=jnp.float32)
      o_ref[...] = o + b2_ref[...]

  return pl.pallas_call(
      body,
      grid=(NBLK,),
      in_specs=[
          pl.BlockSpec((ROWS_BLK, D), lambda i: (i, 0)),
          pl.BlockSpec((D, D), lambda i: (0, 0)),
          pl.BlockSpec((1, D), lambda i: (0, 0)),
          pl.BlockSpec((D, D), lambda i: (0, 0)),
          pl.BlockSpec((1, D), lambda i: (0, 0)),
      ],
      out_specs=pl.BlockSpec((1, D), lambda i: (0, 0)),
      out_shape=jax.ShapeDtypeStruct((1, D), jnp.float32),
      scratch_shapes=[pltpu.VMEM((1, D), jnp.float32)],
  )(h, W1, b1.reshape(1, D), W2, b2.reshape(1, D))


def kernel(x, edge_index, lin0_W, lin0_b, Wm1, bm1, g1, be1, m1, v1,
           Wm2, bm2, g2, be2, m2, v2, lin1_W, lin1_b, lin2_W, lin2_b):
  s1 = g1 / jnp.sqrt(v1 + 1e-5)
  t1 = be1 - m1 * s1
  W1f = Wm1 * s1[:, None, :]
  b1f = bm1 * s1 + t1
  s2 = g2 / jnp.sqrt(v2 + 1e-5)
  t2 = be2 - m2 * s2
  W2f = Wm2 * s2[:, None, :]
  b2f = bm2 * s2 + t2

  pad = EPAD - E
  src_r = jnp.concatenate(
      [edge_index[0], jnp.zeros((pad,), jnp.int32)]).reshape(NC, NS, K, C)
  dst_r = jnp.concatenate(
      [edge_index[1], jnp.full((pad,), N, jnp.int32)]).reshape(NC, NS, K, C)
  zeros = jnp.zeros((NP, D), jnp.float32)

  h = _lin0(x, lin0_W, lin0_b)
  for l in range(L):
    parts = _seg_sum(h, src_r, dst_r, zeros)
    h = _mlp(h, parts[0], parts[1], W1f[l], b1f[l], W2f[l], b2f[l])
  return _readout(h, lin1_W, lin1_b, lin2_W, lin2_b)


# speedup vs baseline: 1.1337x; 1.1337x over previous
import functools

import jax
import jax.numpy as jnp
from jax import lax
from jax.experimental import pallas as pl
from jax.experimental.pallas import tpu as pltpu
from jax.experimental.pallas import tpu_sc as plsc

N = 10000
E = 320000
D = 128
H = 256
L = 4

NC = 2
NS = 16
NW = NC * NS
C = 128
K = 80
EPW = K * C
EPAD = NW * EPW
NBUF = 2
NP = 10008
RPT = 624
REM = N - NS * RPT
ZREM = NP - NS * RPT

ROWS_BLK = 1000
NBLK = N // ROWS_BLK


def _seg_sum(h, src_r, dst_r, zeros):
  mesh = plsc.VectorSubcoreMesh(core_axis_name="c", subcore_axis_name="s")

  @functools.partial(
      pl.kernel,
      out_type=jax.ShapeDtypeStruct((NC, N, D), jnp.float32),
      mesh=mesh,
      scratch_types=[
          pltpu.VMEM((NBUF, C), jnp.int32),
          pltpu.VMEM((NBUF, C), jnp.int32),
          pltpu.VMEM((NBUF, C, D), jnp.float32),
          pltpu.VMEM_SHARED((NP, D), jnp.float32),
          [pltpu.SemaphoreType.DMA] * NBUF,
          [pltpu.SemaphoreType.DMA] * NBUF,
          [pltpu.SemaphoreType.DMA] * NBUF,
          [pltpu.SemaphoreType.DMA] * NBUF,
      ],
  )
  def k(h_hbm, src_hbm, dst_hbm, z_hbm, out_hbm, sidx, didx, rows_v,
        agg_sh, gsem, ssem, sisem, disem):
    c = lax.axis_index("c")
    s = lax.axis_index("s")
    pltpu.sync_copy(z_hbm.at[pl.ds(s * RPT, RPT)],
                    agg_sh.at[pl.ds(s * RPT, RPT)])

    @pl.when(s == 0)
    def _():
      pltpu.sync_copy(z_hbm.at[pl.ds(NS * RPT, ZREM)],
                      agg_sh.at[pl.ds(NS * RPT, ZREM)])

    plsc.subcore_barrier()

    def wait_idx(ref, sem, b):
      pltpu.make_async_copy(src_hbm.at[c, s, 0], ref.at[b], sem[b]).wait()

    def wait_rows(sem, b):
      pltpu.make_async_copy(h_hbm.at[pl.ds(0, C)], rows_v.at[b],
                            sem[b]).wait()

    for b in range(NBUF):
      pltpu.async_copy(src_hbm.at[c, s, b], sidx.at[b], sisem[b])
      pltpu.async_copy(dst_hbm.at[c, s, b], didx.at[b], disem[b])
    for b in range(NBUF):
      wait_idx(sidx, sisem, b)
      pltpu.async_copy(h_hbm.at[sidx.at[b]], rows_v.at[b], gsem[b])

    def body(jp, carry):
      j0 = jp * NBUF
      for b in range(NBUF):
        wait_rows(gsem, b)

        @pl.when(j0 + b + NBUF < K)
        def _():
          pltpu.async_copy(src_hbm.at[c, s, j0 + b + NBUF], sidx.at[b],
                           sisem[b])

        wait_idx(didx, disem, b)
        pltpu.async_copy(rows_v.at[b], agg_sh.at[didx.at[b]], ssem[b],
                         add=True)
      for b in range(NBUF):
        wait_rows(ssem, b)

        @pl.when(j0 + b + NBUF < K)
        def _():
          pltpu.async_copy(dst_hbm.at[c, s, j0 + b + NBUF], didx.at[b],
                           disem[b])
          wait_idx(sidx, sisem, b)
          pltpu.async_copy(h_hbm.at[sidx.at[b]], rows_v.at[b], gsem[b])

      return carry

    lax.fori_loop(0, K // NBUF, body, 0)
    plsc.subcore_barrier()
    pltpu.sync_copy(agg_sh.at[pl.ds(s * RPT, RPT)],
                    out_hbm.at[c, pl.ds(s * RPT, RPT)])

    @pl.when(s == 0)
    def _():
      pltpu.sync_copy(agg_sh.at[pl.ds(NS * RPT, REM)],
                      out_hbm.at[c, pl.ds(NS * RPT, REM)])

  return k(h, src_r, dst_r, zeros)


def _lin0(x, W, b):
  def body(x_ref, w_ref, b_ref, o_ref):
    o_ref[...] = jnp.dot(x_ref[...], w_ref[...],
                         preferred_element_type=jnp.float32) + b_ref[...]

  return pl.pallas_call(
      body,
      grid=(NBLK,),
      in_specs=[
          pl.BlockSpec((ROWS_BLK, D), lambda i: (i, 0)),
          pl.BlockSpec((D, D), lambda i: (0, 0)),
          pl.BlockSpec((1, D), lambda i: (0, 0)),
      ],
      out_specs=pl.BlockSpec((ROWS_BLK, D), lambda i: (i, 0)),
      out_shape=jax.ShapeDtypeStruct((N, D), jnp.float32),
  )(x, W, b.reshape(1, D))


def _mlp(h, p0, p1, W1, b1, W2, b2):
  def body(h_ref, p0_ref, p1_ref, w1_ref, b1_ref, w2_ref, b2_ref, o_ref):
    hh = h_ref[...]
    z = hh + p0_ref[...] + p1_ref[...]
    a = jnp.dot(z, w1_ref[...], preferred_element_type=jnp.float32)
    a = jnp.maximum(a + b1_ref[...], 0.0)
    zz = jnp.dot(a, w2_ref[...], preferred_element_type=jnp.float32)
    o_ref[...] = hh + jnp.maximum(zz + b2_ref[...], 0.0)

  return pl.pallas_call(
      body,
      grid=(NBLK,),
      in_specs=[
          pl.BlockSpec((ROWS_BLK, D), lambda i: (i, 0)),
          pl.BlockSpec((ROWS_BLK, D), lambda i: (i, 0)),
          pl.BlockSpec((ROWS_BLK, D), lambda i: (i, 0)),
          pl.BlockSpec((D, H), lambda i: (0, 0)),
          pl.BlockSpec((1, H), lambda i: (0, 0)),
          pl.BlockSpec((H, D), lambda i: (0, 0)),
          pl.BlockSpec((1, D), lambda i: (0, 0)),
      ],
      out_specs=pl.BlockSpec((ROWS_BLK, D), lambda i: (i, 0)),
      out_shape=jax.ShapeDtypeStruct((N, D), jnp.float32),
  )(h, p0, p1, W1, b1.reshape(1, H), W2, b2.reshape(1, D))


def _readout(h, W1, b1, W2, b2):
  def body(h_ref, w1_ref, b1_ref, w2_ref, b2_ref, o_ref, acc_ref):
    i = pl.program_id(0)

    @pl.when(i == 0)
    def _():
      acc_ref[...] = jnp.zeros_like(acc_ref)

    acc_ref[...] += jnp.sum(h_ref[...], axis=0, keepdims=True)

    @pl.when(i == NBLK - 1)
    def _():
      s = acc_ref[...]
      a = jnp.dot(s, w1_ref[...], preferred_element_type=jnp.float32)
      a = jnp.maximum(a + b1_ref[...], 0.0)
      o = jnp.dot(a, w2_ref[...], preferred_element_type=jnp.float32)
      o_ref[...] = o + b2_ref[...]

  return pl.pallas_call(
      body,
      grid=(NBLK,),
      in_specs=[
          pl.BlockSpec((ROWS_BLK, D), lambda i: (i, 0)),
          pl.BlockSpec((D, D), lambda i: (0, 0)),
          pl.BlockSpec((1, D), lambda i: (0, 0)),
          pl.BlockSpec((D, D), lambda i: (0, 0)),
          pl.BlockSpec((1, D), lambda i: (0, 0)),
      ],
      out_specs=pl.BlockSpec((1, D), lambda i: (0, 0)),
      out_shape=jax.ShapeDtypeStruct((1, D), jnp.float32),
      scratch_shapes=[pltpu.VMEM((1, D), jnp.float32)],
  )(h, W1, b1.reshape(1, D), W2, b2.reshape(1, D))


def kernel(x, edge_index, lin0_W, lin0_b, Wm1, bm1, g1, be1, m1, v1,
           Wm2, bm2, g2, be2, m2, v2, lin1_W, lin1_b, lin2_W, lin2_b):
  s1 = g1 / jnp.sqrt(v1 + 1e-5)
  t1 = be1 - m1 * s1
  W1f = Wm1 * s1[:, None, :]
  b1f = bm1 * s1 + t1
  s2 = g2 / jnp.sqrt(v2 + 1e-5)
  t2 = be2 - m2 * s2
  W2f = Wm2 * s2[:, None, :]
  b2f = bm2 * s2 + t2

  ppw = EPW - E // NW
  pad_src = jnp.zeros((NW, ppw), jnp.int32)
  pad_dst = jnp.broadcast_to(N + jnp.arange(ppw, dtype=jnp.int32) % 8,
                             (NW, ppw))
  src_r = jnp.concatenate(
      [edge_index[0].reshape(NW, -1), pad_src], axis=1).reshape(NC, NS, K, C)
  dst_r = jnp.concatenate(
      [edge_index[1].reshape(NW, -1), pad_dst], axis=1).reshape(NC, NS, K, C)
  zeros = jnp.zeros((NP, D), jnp.float32)

  h = _lin0(x, lin0_W, lin0_b)
  for l in range(L):
    parts = _seg_sum(h, src_r, dst_r, zeros)
    h = _mlp(h, parts[0], parts[1], W1f[l], b1f[l], W2f[l], b2f[l])
  return _readout(h, lin1_W, lin1_b, lin2_W, lin2_b)


# speedup vs baseline: 2.9575x; 2.6087x over previous
import functools

import jax
import jax.numpy as jnp
from jax import lax
from jax.experimental import pallas as pl
from jax.experimental.pallas import tpu as pltpu
from jax.experimental.pallas import tpu_sc as plsc

N = 10000
E = 320000
D = 128
H = 256
L = 4

NC = 2
NS = 16
NW = NC * NS
EPW = E // NW
C = 125
K = EPW // C
NBUF = 2
RPT = 624
REM = N - NS * RPT

ROWS_BLK = 1000
NBLK = N // ROWS_BLK


def _seg_sum(h, src_r, dst_r, zeros):
  mesh = plsc.VectorSubcoreMesh(core_axis_name="c", subcore_axis_name="s")

  @functools.partial(
      pl.kernel,
      out_type=jax.ShapeDtypeStruct((NC, N, D), jnp.float32),
      mesh=mesh,
      scratch_types=[
          pltpu.VMEM((NBUF, C), jnp.int32),
          pltpu.VMEM((NBUF, C), jnp.int32),
          pltpu.VMEM((NBUF, C, D), jnp.float32),
          pltpu.VMEM_SHARED((N, D), jnp.float32),
          [pltpu.SemaphoreType.DMA] * NBUF,
          [pltpu.SemaphoreType.DMA] * NBUF,
          [pltpu.SemaphoreType.DMA] * NBUF,
          [pltpu.SemaphoreType.DMA] * NBUF,
      ],
  )
  def k(h_hbm, src_hbm, dst_hbm, z_hbm, out_hbm, sidx, didx, rows_v,
        agg_sh, gsem, ssem, sisem, disem):
    c = lax.axis_index("c")
    s = lax.axis_index("s")
    pltpu.sync_copy(z_hbm.at[pl.ds(s * RPT, RPT)],
                    agg_sh.at[pl.ds(s * RPT, RPT)])

    @pl.when(s == 0)
    def _():
      pltpu.sync_copy(z_hbm.at[pl.ds(NS * RPT, REM)],
                      agg_sh.at[pl.ds(NS * RPT, REM)])

    plsc.subcore_barrier()

    def wait_idx(ref, sem, b):
      pltpu.make_async_copy(src_hbm.at[c, s, 0], ref.at[b], sem[b]).wait()

    def wait_rows(sem, b):
      pltpu.make_async_copy(h_hbm.at[sidx.at[b]], rows_v.at[b],
                            sem[b]).wait()

    for b in range(NBUF):
      pltpu.async_copy(src_hbm.at[c, s, b], sidx.at[b], sisem[b])
      pltpu.async_copy(dst_hbm.at[c, s, b], didx.at[b], disem[b])
    for b in range(NBUF):
      wait_idx(sidx, sisem, b)
      pltpu.async_copy(h_hbm.at[sidx.at[b]], rows_v.at[b], gsem[b])

    def body(jp, carry):
      j0 = jp * NBUF
      for b in range(NBUF):
        wait_rows(gsem, b)

        @pl.when(j0 + b + NBUF < K)
        def _():
          pltpu.async_copy(src_hbm.at[c, s, j0 + b + NBUF], sidx.at[b],
                           sisem[b])

        wait_idx(didx, disem, b)
        pltpu.async_copy(rows_v.at[b], agg_sh.at[didx.at[b]], ssem[b],
                         add=True)
      for b in range(NBUF):
        wait_rows(ssem, b)

        @pl.when(j0 + b + NBUF < K)
        def _():
          pltpu.async_copy(dst_hbm.at[c, s, j0 + b + NBUF], didx.at[b],
                           disem[b])
          wait_idx(sidx, sisem, b)
          pltpu.async_copy(h_hbm.at[sidx.at[b]], rows_v.at[b], gsem[b])

      return carry

    lax.fori_loop(0, K // NBUF, body, 0)
    plsc.subcore_barrier()
    pltpu.sync_copy(agg_sh.at[pl.ds(s * RPT, RPT)],
                    out_hbm.at[c, pl.ds(s * RPT, RPT)])

    @pl.when(s == 0)
    def _():
      pltpu.sync_copy(agg_sh.at[pl.ds(NS * RPT, REM)],
                      out_hbm.at[c, pl.ds(NS * RPT, REM)])

  return k(h, src_r, dst_r, zeros)


def _lin0(x, W, b):
  def body(x_ref, w_ref, b_ref, o_ref):
    o_ref[...] = jnp.dot(x_ref[...], w_ref[...],
                         preferred_element_type=jnp.float32) + b_ref[...]

  return pl.pallas_call(
      body,
      grid=(NBLK,),
      in_specs=[
          pl.BlockSpec((ROWS_BLK, D), lambda i: (i, 0)),
          pl.BlockSpec((D, D), lambda i: (0, 0)),
          pl.BlockSpec((1, D), lambda i: (0, 0)),
      ],
      out_specs=pl.BlockSpec((ROWS_BLK, D), lambda i: (i, 0)),
      out_shape=jax.ShapeDtypeStruct((N, D), jnp.float32),
  )(x, W, b.reshape(1, D))


def _mlp(h, p0, p1, W1, b1, W2, b2):
  def body(h_ref, p0_ref, p1_ref, w1_ref, b1_ref, w2_ref, b2_ref, o_ref):
    hh = h_ref[...]
    z = hh + p0_ref[...] + p1_ref[...]
    a = jnp.dot(z, w1_ref[...], preferred_element_type=jnp.float32)
    a = jnp.maximum(a + b1_ref[...], 0.0)
    zz = jnp.dot(a, w2_ref[...], preferred_element_type=jnp.float32)
    o_ref[...] = hh + jnp.maximum(zz + b2_ref[...], 0.0)

  return pl.pallas_call(
      body,
      grid=(NBLK,),
      in_specs=[
          pl.BlockSpec((ROWS_BLK, D), lambda i: (i, 0)),
          pl.BlockSpec((ROWS_BLK, D), lambda i: (i, 0)),
          pl.BlockSpec((ROWS_BLK, D), lambda i: (i, 0)),
          pl.BlockSpec((D, H), lambda i: (0, 0)),
          pl.BlockSpec((1, H), lambda i: (0, 0)),
          pl.BlockSpec((H, D), lambda i: (0, 0)),
          pl.BlockSpec((1, D), lambda i: (0, 0)),
      ],
      out_specs=pl.BlockSpec((ROWS_BLK, D), lambda i: (i, 0)),
      out_shape=jax.ShapeDtypeStruct((N, D), jnp.float32),
  )(h, p0, p1, W1, b1.reshape(1, H), W2, b2.reshape(1, D))


def _readout(h, W1, b1, W2, b2):
  def body(h_ref, w1_ref, b1_ref, w2_ref, b2_ref, o_ref, acc_ref):
    i = pl.program_id(0)

    @pl.when(i == 0)
    def _():
      acc_ref[...] = jnp.zeros_like(acc_ref)

    acc_ref[...] += jnp.sum(h_ref[...], axis=0, keepdims=True)

    @pl.when(i == NBLK - 1)
    def _():
      s = acc_ref[...]
      a = jnp.dot(s, w1_ref[...], preferred_element_type=jnp.float32)
      a = jnp.maximum(a + b1_ref[...], 0.0)
      o = jnp.dot(a, w2_ref[...], preferred_element_type=jnp.float32)
      o_ref[...] = o + b2_ref[...]

  return pl.pallas_call(
      body,
      grid=(NBLK,),
      in_specs=[
          pl.BlockSpec((ROWS_BLK, D), lambda i: (i, 0)),
          pl.BlockSpec((D, D), lambda i: (0, 0)),
          pl.BlockSpec((1, D), lambda i: (0, 0)),
          pl.BlockSpec((D, D), lambda i: (0, 0)),
          pl.BlockSpec((1, D), lambda i: (0, 0)),
      ],
      out_specs=pl.BlockSpec((1, D), lambda i: (0, 0)),
      out_shape=jax.ShapeDtypeStruct((1, D), jnp.float32),
      scratch_shapes=[pltpu.VMEM((1, D), jnp.float32)],
  )(h, W1, b1.reshape(1, D), W2, b2.reshape(1, D))


def kernel(x, edge_index, lin0_W, lin0_b, Wm1, bm1, g1, be1, m1, v1,
           Wm2, bm2, g2, be2, m2, v2, lin1_W, lin1_b, lin2_W, lin2_b):
  s1 = g1 / jnp.sqrt(v1 + 1e-5)
  t1 = be1 - m1 * s1
  W1f = Wm1 * s1[:, None, :]
  b1f = bm1 * s1 + t1
  s2 = g2 / jnp.sqrt(v2 + 1e-5)
  t2 = be2 - m2 * s2
  W2f = Wm2 * s2[:, None, :]
  b2f = bm2 * s2 + t2

  src_r = edge_index[0].reshape(NC, NS, K, C)
  dst_r = edge_index[1].reshape(NC, NS, K, C)
  zeros = jnp.zeros((N, D), jnp.float32)

  h = _lin0(x, lin0_W, lin0_b)
  for l in range(L):
    parts = _seg_sum(h, src_r, dst_r, zeros)
    h = _mlp(h, parts[0], parts[1], W1f[l], b1f[l], W2f[l], b2f[l])
  return _readout(h, lin1_W, lin1_b, lin2_W, lin2_b)


# speedup vs baseline: 3.2958x; 1.1144x over previous
import functools

import jax
import jax.numpy as jnp
from jax import lax
from jax.experimental import pallas as pl
from jax.experimental.pallas import tpu as pltpu
from jax.experimental.pallas import tpu_sc as plsc

N = 10000
E = 320000
D = 128
H = 256
L = 4

NC = 2
NS = 16
NW = NC * NS
EPW = E // NW
C = 100
K = EPW // C
NBUF = 3
KTAIL = (K // NBUF) * NBUF
RPT = 624
REM = N - NS * RPT

ROWS_BLK = 1000
NBLK = N // ROWS_BLK


def _seg_sum(h, src_r, dst_r, zeros):
  mesh = plsc.VectorSubcoreMesh(core_axis_name="c", subcore_axis_name="s")

  @functools.partial(
      pl.kernel,
      out_type=jax.ShapeDtypeStruct((NC, N, D), jnp.float32),
      mesh=mesh,
      scratch_types=[
          pltpu.VMEM((NBUF, C), jnp.int32),
          pltpu.VMEM((NBUF, C), jnp.int32),
          pltpu.VMEM((NBUF, C, D), jnp.float32),
          pltpu.VMEM_SHARED((N, D), jnp.float32),
          [pltpu.SemaphoreType.DMA] * NBUF,
          [pltpu.SemaphoreType.DMA] * NBUF,
          [pltpu.SemaphoreType.DMA] * NBUF,
          [pltpu.SemaphoreType.DMA] * NBUF,
      ],
  )
  def k(h_hbm, src_hbm, dst_hbm, z_hbm, out_hbm, sidx, didx, rows_v,
        agg_sh, gsem, ssem, sisem, disem):
    c = lax.axis_index("c")
    s = lax.axis_index("s")
    pltpu.sync_copy(z_hbm.at[pl.ds(s * RPT, RPT)],
                    agg_sh.at[pl.ds(s * RPT, RPT)])

    @pl.when(s == 0)
    def _():
      pltpu.sync_copy(z_hbm.at[pl.ds(NS * RPT, REM)],
                      agg_sh.at[pl.ds(NS * RPT, REM)])

    plsc.subcore_barrier()

    def wait_idx(ref, sem, b):
      pltpu.make_async_copy(src_hbm.at[c, s, 0], ref.at[b], sem[b]).wait()

    def wait_rows(sem, b):
      pltpu.make_async_copy(h_hbm.at[sidx.at[b]], rows_v.at[b],
                            sem[b]).wait()

    for b in range(NBUF):
      pltpu.async_copy(src_hbm.at[c, s, b], sidx.at[b], sisem[b])
      pltpu.async_copy(dst_hbm.at[c, s, b], didx.at[b], disem[b])
    for b in range(NBUF):
      wait_idx(sidx, sisem, b)
      pltpu.async_copy(h_hbm.at[sidx.at[b]], rows_v.at[b], gsem[b])

    def body(jp, carry):
      j0 = jp * NBUF
      for b in range(NBUF):
        wait_rows(gsem, b)

        @pl.when(j0 + b + NBUF < K)
        def _():
          pltpu.async_copy(src_hbm.at[c, s, j0 + b + NBUF], sidx.at[b],
                           sisem[b])

        wait_idx(didx, disem, b)
        pltpu.async_copy(rows_v.at[b], agg_sh.at[didx.at[b]], ssem[b],
                         add=True)
      for b in range(NBUF):
        wait_rows(ssem, b)

        @pl.when(j0 + b + NBUF < K)
        def _():
          pltpu.async_copy(dst_hbm.at[c, s, j0 + b + NBUF], didx.at[b],
                           disem[b])
          wait_idx(sidx, sisem, b)
          pltpu.async_copy(h_hbm.at[sidx.at[b]], rows_v.at[b], gsem[b])

      return carry

    lax.fori_loop(0, K // NBUF, body, 0)
    for j in range(KTAIL, K):
      b = j % NBUF
      wait_rows(gsem, b)
      wait_idx(didx, disem, b)
      pltpu.async_copy(rows_v.at[b], agg_sh.at[didx.at[b]], ssem[b],
                       add=True)
    for j in range(KTAIL, K):
      wait_rows(ssem, j % NBUF)
    plsc.subcore_barrier()
    pltpu.sync_copy(agg_sh.at[pl.ds(s * RPT, RPT)],
                    out_hbm.at[c, pl.ds(s * RPT, RPT)])

    @pl.when(s == 0)
    def _():
      pltpu.sync_copy(agg_sh.at[pl.ds(NS * RPT, REM)],
                      out_hbm.at[c, pl.ds(NS * RPT, REM)])

  return k(h, src_r, dst_r, zeros)


def _lin0(x, W, b):
  def body(x_ref, w_ref, b_ref, o_ref):
    o_ref[...] = jnp.dot(x_ref[...], w_ref[...],
                         preferred_element_type=jnp.float32) + b_ref[...]

  return pl.pallas_call(
      body,
      grid=(NBLK,),
      in_specs=[
          pl.BlockSpec((ROWS_BLK, D), lambda i: (i, 0)),
          pl.BlockSpec((D, D), lambda i: (0, 0)),
          pl.BlockSpec((1, D), lambda i: (0, 0)),
      ],
      out_specs=pl.BlockSpec((ROWS_BLK, D), lambda i: (i, 0)),
      out_shape=jax.ShapeDtypeStruct((N, D), jnp.float32),
  )(x, W, b.reshape(1, D))


def _mlp(h, p0, p1, W1, b1, W2, b2):
  def body(h_ref, p0_ref, p1_ref, w1_ref, b1_ref, w2_ref, b2_ref, o_ref):
    hh = h_ref[...]
    z = hh + p0_ref[...] + p1_ref[...]
    a = jnp.dot(z, w1_ref[...], preferred_element_type=jnp.float32)
    a = jnp.maximum(a + b1_ref[...], 0.0)
    zz = jnp.dot(a, w2_ref[...], preferred_element_type=jnp.float32)
    o_ref[...] = hh + jnp.maximum(zz + b2_ref[...], 0.0)

  return pl.pallas_call(
      body,
      grid=(NBLK,),
      in_specs=[
          pl.BlockSpec((ROWS_BLK, D), lambda i: (i, 0)),
          pl.BlockSpec((ROWS_BLK, D), lambda i: (i, 0)),
          pl.BlockSpec((ROWS_BLK, D), lambda i: (i, 0)),
          pl.BlockSpec((D, H), lambda i: (0, 0)),
          pl.BlockSpec((1, H), lambda i: (0, 0)),
          pl.BlockSpec((H, D), lambda i: (0, 0)),
          pl.BlockSpec((1, D), lambda i: (0, 0)),
      ],
      out_specs=pl.BlockSpec((ROWS_BLK, D), lambda i: (i, 0)),
      out_shape=jax.ShapeDtypeStruct((N, D), jnp.float32),
  )(h, p0, p1, W1, b1.reshape(1, H), W2, b2.reshape(1, D))


def _readout(h, W1, b1, W2, b2):
  def body(h_ref, w1_ref, b1_ref, w2_ref, b2_ref, o_ref, acc_ref):
    i = pl.program_id(0)

    @pl.when(i == 0)
    def _():
      acc_ref[...] = jnp.zeros_like(acc_ref)

    acc_ref[...] += jnp.sum(h_ref[...], axis=0, keepdims=True)

    @pl.when(i == NBLK - 1)
    def _():
      s = acc_ref[...]
      a = jnp.dot(s, w1_ref[...], preferred_element_type=jnp.float32)
      a = jnp.maximum(a + b1_ref[...], 0.0)
      o = jnp.dot(a, w2_ref[...], preferred_element_type=jnp.float32)
      o_ref[...] = o + b2_ref[...]

  return pl.pallas_call(
      body,
      grid=(NBLK,),
      in_specs=[
          pl.BlockSpec((ROWS_BLK, D), lambda i: (i, 0)),
          pl.BlockSpec((D, D), lambda i: (0, 0)),
          pl.BlockSpec((1, D), lambda i: (0, 0)),
          pl.BlockSpec((D, D), lambda i: (0, 0)),
          pl.BlockSpec((1, D), lambda i: (0, 0)),
      ],
      out_specs=pl.BlockSpec((1, D), lambda i: (0, 0)),
      out_shape=jax.ShapeDtypeStruct((1, D), jnp.float32),
      scratch_shapes=[pltpu.VMEM((1, D), jnp.float32)],
  )(h, W1, b1.reshape(1, D), W2, b2.reshape(1, D))


def kernel(x, edge_index, lin0_W, lin0_b, Wm1, bm1, g1, be1, m1, v1,
           Wm2, bm2, g2, be2, m2, v2, lin1_W, lin1_b, lin2_W, lin2_b):
  s1 = g1 / jnp.sqrt(v1 + 1e-5)
  t1 = be1 - m1 * s1
  W1f = Wm1 * s1[:, None, :]
  b1f = bm1 * s1 + t1
  s2 = g2 / jnp.sqrt(v2 + 1e-5)
  t2 = be2 - m2 * s2
  W2f = Wm2 * s2[:, None, :]
  b2f = bm2 * s2 + t2

  src_r = edge_index[0].reshape(NC, NS, K, C)
  dst_r = edge_index[1].reshape(NC, NS, K, C)
  zeros = jnp.zeros((N, D), jnp.float32)

  h = _lin0(x, lin0_W, lin0_b)
  for l in range(L):
    parts = _seg_sum(h, src_r, dst_r, zeros)
    h = _mlp(h, parts[0], parts[1], W1f[l], b1f[l], W2f[l], b2f[l])
  return _readout(h, lin1_W, lin1_b, lin2_W, lin2_b)
